# baseline (speedup 1.0000x reference)
import functools

import jax
import jax.numpy as jnp
from jax import lax
from jax.experimental import pallas as pl
from jax.experimental.pallas import tpu as pltpu
from jax.experimental.pallas import tpu_sc as plsc

N_ATOMS = 10000
N_BONDS = 320000
MAX_NB = 32
ATOM_FDIM = 133
BOND_FDIM = 147
HIDDEN = 128
N_MOLS = 500

NC = 2
NS = 16
NW = NC * NS
LANES = 16

NA_PAD = 10240
CA = 64
NCHUNK_A = NA_PAD // CA
NGRP = 4

CB = 512
NCHUNK_B = N_BONDS // CB
KS = CB // 128

BR = 1280
DR = 1000


def _sc_mesh():
    return plsc.VectorSubcoreMesh(
        core_axis_name="c", subcore_axis_name="s", num_cores=NC, num_subcores=NS
    )


def _make_gather_sum(negate: bool):
    def body(table_hbm, a2bS_hbm, out_hbm, idx_v, acc_v, sem0, sem1):
        wid = lax.axis_index("s") * NC + lax.axis_index("c")

        def chunk(ci, carry):
            cid = wid + ci * NW

            @pl.when(cid < NCHUNK_A)
            def _():
                abase = cid * CA
                pltpu.sync_copy(a2bS_hbm.at[cid], idx_v)
                leaders = [
                    pltpu.async_copy(
                        table_hbm.at[idx_v.at[g * 8]],
                        acc_v.at[pl.ds(g * CA, CA)],
                        sem0,
                    )
                    for g in range(NGRP)
                ]
                for d in leaders:
                    d.wait()
                descs = [
                    pltpu.async_copy(
                        table_hbm.at[idx_v.at[g * 8 + 1 + j]],
                        acc_v.at[pl.ds(g * CA, CA)],
                        sem1,
                        add=True,
                    )
                    for g in range(NGRP)
                    for j in range(7)
                ]
                for d in descs:
                    d.wait()

                sgn = -1.0 if negate else 1.0

                def red_row(i, c3):
                    for j in range(8):
                        sl = pl.ds(j * LANES, LANES)
                        v = acc_v[i, sl]
                        for g in range(1, NGRP):
                            v = v + acc_v[g * CA + i, sl]
                        acc_v[i, sl] = sgn * v
                    return c3

                lax.fori_loop(0, CA, red_row, 0)

                pltpu.sync_copy(
                    acc_v.at[pl.ds(0, CA)], out_hbm.at[pl.ds(abase, CA)]
                )

            return carry

        lax.fori_loop(0, (NCHUNK_A + NW - 1) // NW, chunk, 0)

    return pl.kernel(
        body,
        out_type=jax.ShapeDtypeStruct((NA_PAD, HIDDEN), jnp.float32),
        mesh=_sc_mesh(),
        scratch_types=[
            pltpu.VMEM((MAX_NB, CA), jnp.int32),
            pltpu.VMEM((NGRP * CA, HIDDEN), jnp.float32),
            pltpu.SemaphoreType.DMA,
            pltpu.SemaphoreType.DMA,
        ],
    )


def _make_combine(relu: bool):
    def body(inp_hbm, atab_hbm, ntab_hbm, b2a_hbm, b2revb_hbm, out_hbm,
             idxa_v, idxr_v, buf_v, sem):
        wid = lax.axis_index("s") * NC + lax.axis_index("c")

        def chunk(ci, carry):
            cid = wid + ci * NW

            @pl.when(cid < NCHUNK_B)
            def _():
                base = cid * CB
                pltpu.sync_copy(b2a_hbm.at[pl.ds(base, CB)], idxa_v)
                pltpu.sync_copy(b2revb_hbm.at[pl.ds(base, CB)], idxr_v)
                pltpu.sync_copy(inp_hbm.at[pl.ds(base, CB)], buf_v)
                descs = []
                for j in range(KS):
                    descs.append(
                        pltpu.async_copy(
                            atab_hbm.at[idxa_v.at[pl.ds(j * 128, 128)]],
                            buf_v.at[pl.ds(j * 128, 128)],
                            sem,
                            add=True,
                        )
                    )
                for j in range(KS):
                    descs.append(
                        pltpu.async_copy(
                            ntab_hbm.at[idxr_v.at[pl.ds(j * 128, 128)]],
                            buf_v.at[pl.ds(j * 128, 128)],
                            sem,
                            add=True,
                        )
                    )
                for d in descs:
                    d.wait()

                if relu:
                    def relu_row(i, c2):
                        for j in range(8):
                            sl = pl.ds(j * LANES, LANES)
                            buf_v[i, sl] = jnp.maximum(buf_v[i, sl], 0.0)
                        return c2

                    lax.fori_loop(0, CB, relu_row, 0)

                pltpu.sync_copy(buf_v, out_hbm.at[pl.ds(base, CB)])

            return carry

        lax.fori_loop(0, (NCHUNK_B + NW - 1) // NW, chunk, 0)

    return pl.kernel(
        body,
        out_type=jax.ShapeDtypeStruct((N_BONDS, HIDDEN), jnp.float32),
        mesh=_sc_mesh(),
        scratch_types=[
            pltpu.VMEM((CB,), jnp.int32),
            pltpu.VMEM((CB,), jnp.int32),
            pltpu.VMEM((CB, HIDDEN), jnp.float32),
            pltpu.SemaphoreType.DMA,
        ],
    )


def _tc_init(f_bonds, W_i, W_h):
    def body(fb_ref, wi_ref, wh_ref, inp_ref, nmh_ref):
        inp = jnp.dot(fb_ref[...], wi_ref[...], preferred_element_type=jnp.float32)
        inp_ref[...] = inp
        msg = jnp.maximum(inp, 0.0)
        nmh_ref[...] = -jnp.dot(msg, wh_ref[...], preferred_element_type=jnp.float32)

    return pl.pallas_call(
        body,
        grid=(N_BONDS // BR,),
        in_specs=[
            pl.BlockSpec((BR, BOND_FDIM), lambda i: (i, 0)),
            pl.BlockSpec((BOND_FDIM, HIDDEN), lambda i: (0, 0)),
            pl.BlockSpec((HIDDEN, HIDDEN), lambda i: (0, 0)),
        ],
        out_specs=[
            pl.BlockSpec((BR, HIDDEN), lambda i: (i, 0)),
            pl.BlockSpec((BR, HIDDEN), lambda i: (i, 0)),
        ],
        out_shape=[
            jax.ShapeDtypeStruct((N_BONDS, HIDDEN), jnp.float32),
            jax.ShapeDtypeStruct((N_BONDS, HIDDEN), jnp.float32),
        ],
    )(f_bonds, W_i, W_h)


def _tc_step(premsg, W_h):
    def body(pm_ref, wh_ref, nmh_ref):
        msg = jnp.maximum(pm_ref[...], 0.0)
        nmh_ref[...] = -jnp.dot(msg, wh_ref[...], preferred_element_type=jnp.float32)

    return pl.pallas_call(
        body,
        grid=(N_BONDS // BR,),
        in_specs=[
            pl.BlockSpec((BR, HIDDEN), lambda i: (i, 0)),
            pl.BlockSpec((HIDDEN, HIDDEN), lambda i: (0, 0)),
        ],
        out_specs=pl.BlockSpec((BR, HIDDEN), lambda i: (i, 0)),
        out_shape=jax.ShapeDtypeStruct((N_BONDS, HIDDEN), jnp.float32),
    )(premsg, W_h)


def _tc_readout(f_atoms, amsg, Wo_a, Wo_m, b_o2, seg2d):
    n_blocks = N_ATOMS // DR

    def body(fa_ref, am_ref, seg_ref, woa_ref, wom_ref, bo_ref, out_ref,
             sums_v, cnts_v):
        k = pl.program_id(0)

        @pl.when(k == 0)
        def _():
            sums_v[...] = jnp.zeros_like(sums_v)
            cnts_v[...] = jnp.zeros_like(cnts_v)

        hid = jnp.dot(fa_ref[...], woa_ref[...], preferred_element_type=jnp.float32)
        hid += jnp.dot(am_ref[...], wom_ref[...], preferred_element_type=jnp.float32)
        hid = jnp.maximum(hid + bo_ref[...], 0.0)
        rows = lax.broadcasted_iota(jnp.int32, (N_MOLS, DR), 0)
        oh = (rows == seg_ref[0]).astype(jnp.float32)
        sums_v[...] += jnp.dot(oh, hid, preferred_element_type=jnp.float32)
        cnts_v[...] += jnp.sum(oh, axis=1, keepdims=True)

        @pl.when(k == n_blocks - 1)
        def _():
            out_ref[...] = sums_v[...] / jnp.maximum(cnts_v[...], 1.0)

    return pl.pallas_call(
        body,
        grid=(n_blocks,),
        in_specs=[
            pl.BlockSpec((DR, ATOM_FDIM), lambda k: (k, 0)),
            pl.BlockSpec((DR, HIDDEN), lambda k: (k, 0)),
            pl.BlockSpec((1, 1, DR), lambda k: (k, 0, 0)),
            pl.BlockSpec((ATOM_FDIM, HIDDEN), lambda k: (0, 0)),
            pl.BlockSpec((HIDDEN, HIDDEN), lambda k: (0, 0)),
            pl.BlockSpec((1, HIDDEN), lambda k: (0, 0)),
        ],
        out_specs=pl.BlockSpec((N_MOLS, HIDDEN), lambda k: (0, 0)),
        out_shape=jax.ShapeDtypeStruct((N_MOLS, HIDDEN), jnp.float32),
        scratch_shapes=[
            pltpu.VMEM((N_MOLS, HIDDEN), jnp.float32),
            pltpu.VMEM((N_MOLS, 1), jnp.float32),
        ],
    )(f_atoms, amsg, seg2d, Wo_a, Wo_m, b_o2)


def _sc_tiny(b_o2):
    def body(x_hbm, y_hbm, buf_v):
        wid = lax.axis_index("s") * NC + lax.axis_index("c")

        @pl.when(wid == 0)
        def _():
            pltpu.sync_copy(x_hbm, buf_v)
            pltpu.sync_copy(buf_v, y_hbm)

    return pl.kernel(
        body,
        out_type=jax.ShapeDtypeStruct((1, HIDDEN), jnp.float32),
        mesh=_sc_mesh(),
        scratch_types=[pltpu.VMEM((1, HIDDEN), jnp.float32)],
    )(b_o2)


def kernel(f_atoms, f_bonds, W_i, W_h, W_o, b_o, a2b, b2a, b2revb, segment_ids):
    a2b = a2b.astype(jnp.int32)
    b2a = b2a.astype(jnp.int32)
    b2revb = b2revb.astype(jnp.int32)
    segment_ids = segment_ids.astype(jnp.int32)

    a2bS = (
        jnp.pad(a2b, ((0, NA_PAD - N_ATOMS), (0, 0)))
        .reshape(NCHUNK_A, CA, MAX_NB)
        .transpose(0, 2, 1)
    )
    Wo_a = W_o[:ATOM_FDIM]
    Wo_m = W_o[ATOM_FDIM:]
    b_o2 = _sc_tiny(b_o.reshape(1, HIDDEN))
    seg2d = segment_ids.reshape(N_ATOMS // DR, 1, DR)

    gather_neg = _make_gather_sum(negate=True)
    gather_plain = _make_gather_sum(negate=False)
    combine = _make_combine(relu=False)
    combine_relu = _make_combine(relu=True)

    inp, nmh1 = _tc_init(f_bonds, W_i, W_h)
    a1 = gather_neg(nmh1, a2bS)
    premsg2 = combine(inp, a1, nmh1, b2a, b2revb)
    nmh2 = _tc_step(premsg2, W_h)
    a2 = gather_neg(nmh2, a2bS)
    msg3 = combine_relu(inp, a2, nmh2, b2a, b2revb)
    amsg3 = gather_plain(msg3, a2bS)
    return _tc_readout(f_atoms, amsg3[:N_ATOMS], Wo_a, Wo_m, b_o2, seg2d)


# speedup vs baseline: 1.5434x; 1.5434x over previous
import functools

import jax
import jax.numpy as jnp
from jax import lax
from jax.experimental import pallas as pl
from jax.experimental.pallas import tpu as pltpu
from jax.experimental.pallas import tpu_sc as plsc

N_ATOMS = 10000
N_BONDS = 320000
MAX_NB = 32
ATOM_FDIM = 133
BOND_FDIM = 147
HIDDEN = 128
N_MOLS = 500

NC = 2
NS = 16
NW = NC * NS
LANES = 16

NA_PAD = 10240
CA = 64
NCHUNK_A = NA_PAD // CA
NGRP = 4

CB = 512
NCHUNK_B = N_BONDS // CB
KS = CB // 128

BR = 1280
DR = 1000


def _sc_mesh():
    return plsc.VectorSubcoreMesh(
        core_axis_name="c", subcore_axis_name="s", num_cores=NC, num_subcores=NS
    )


def _make_gather_sum(negate: bool):
    def body(table_hbm, a2bS_hbm, out_hbm, idx_v, acc_v, sem0, sem1):
        wid = lax.axis_index("s") * NC + lax.axis_index("c")

        def chunk(ci, carry):
            cid = wid + ci * NW

            @pl.when(cid < NCHUNK_A)
            def _():
                abase = cid * CA
                pltpu.sync_copy(a2bS_hbm.at[cid], idx_v)
                leaders = [
                    pltpu.async_copy(
                        table_hbm.at[idx_v.at[g * 8]],
                        acc_v.at[pl.ds(g * CA, CA)],
                        sem0,
                    )
                    for g in range(NGRP)
                ]
                for d in leaders:
                    d.wait()
                descs = [
                    pltpu.async_copy(
                        table_hbm.at[idx_v.at[g * 8 + 1 + j]],
                        acc_v.at[pl.ds(g * CA, CA)],
                        sem1,
                        add=True,
                    )
                    for g in range(NGRP)
                    for j in range(7)
                ]
                for d in descs:
                    d.wait()

                sgn = -1.0 if negate else 1.0

                def red_row(i, c3):
                    for j in range(8):
                        sl = pl.ds(j * LANES, LANES)
                        v = acc_v[i, sl]
                        for g in range(1, NGRP):
                            v = v + acc_v[g * CA + i, sl]
                        acc_v[i, sl] = sgn * v
                    return c3

                lax.fori_loop(0, CA, red_row, 0)

                pltpu.sync_copy(
                    acc_v.at[pl.ds(0, CA)], out_hbm.at[pl.ds(abase, CA)]
                )

            return carry

        lax.fori_loop(0, (NCHUNK_A + NW - 1) // NW, chunk, 0)

    return pl.kernel(
        body,
        out_type=jax.ShapeDtypeStruct((NA_PAD, HIDDEN), jnp.float32),
        mesh=_sc_mesh(),
        scratch_types=[
            pltpu.VMEM((MAX_NB, CA), jnp.int32),
            pltpu.VMEM((NGRP * CA, HIDDEN), jnp.float32),
            pltpu.SemaphoreType.DMA,
            pltpu.SemaphoreType.DMA,
        ],
    )


def _make_combine(relu: bool):
    def body(inp_hbm, atab_hbm, ntab_hbm, b2a_hbm, b2revb_hbm, out_hbm,
             idxa_v, idxr_v, buf_v, sem):
        wid = lax.axis_index("s") * NC + lax.axis_index("c")

        def chunk(ci, carry):
            cid = wid + ci * NW

            @pl.when(cid < NCHUNK_B)
            def _():
                base = cid * CB
                pltpu.sync_copy(b2a_hbm.at[pl.ds(base, CB)], idxa_v)
                pltpu.sync_copy(b2revb_hbm.at[pl.ds(base, CB)], idxr_v)
                pltpu.sync_copy(inp_hbm.at[pl.ds(base, CB)], buf_v)
                descs = []
                for j in range(KS):
                    descs.append(
                        pltpu.async_copy(
                            atab_hbm.at[idxa_v.at[pl.ds(j * 128, 128)]],
                            buf_v.at[pl.ds(j * 128, 128)],
                            sem,
                            add=True,
                        )
                    )
                for j in range(KS):
                    descs.append(
                        pltpu.async_copy(
                            ntab_hbm.at[idxr_v.at[pl.ds(j * 128, 128)]],
                            buf_v.at[pl.ds(j * 128, 128)],
                            sem,
                            add=True,
                        )
                    )
                for d in descs:
                    d.wait()

                if relu:
                    def relu_row(i, c2):
                        for j in range(8):
                            sl = pl.ds(j * LANES, LANES)
                            buf_v[i, sl] = jnp.maximum(buf_v[i, sl], 0.0)
                        return c2

                    lax.fori_loop(0, CB, relu_row, 0)

                pltpu.sync_copy(buf_v, out_hbm.at[pl.ds(base, CB)])

            return carry

        lax.fori_loop(0, (NCHUNK_B + NW - 1) // NW, chunk, 0)

    return pl.kernel(
        body,
        out_type=jax.ShapeDtypeStruct((N_BONDS, HIDDEN), jnp.float32),
        mesh=_sc_mesh(),
        scratch_types=[
            pltpu.VMEM((CB,), jnp.int32),
            pltpu.VMEM((CB,), jnp.int32),
            pltpu.VMEM((CB, HIDDEN), jnp.float32),
            pltpu.SemaphoreType.DMA,
        ],
    )


def _tc_init(f_bonds, W_i, W_h):
    def body(fb_ref, wi_ref, wh_ref, inp_ref, nmh_ref):
        inp = jnp.dot(fb_ref[...], wi_ref[...], preferred_element_type=jnp.float32)
        inp_ref[...] = inp
        msg = jnp.maximum(inp, 0.0)
        nmh_ref[...] = -jnp.dot(msg, wh_ref[...], preferred_element_type=jnp.float32)

    return pl.pallas_call(
        body,
        grid=(N_BONDS // BR,),
        in_specs=[
            pl.BlockSpec((BR, BOND_FDIM), lambda i: (i, 0)),
            pl.BlockSpec((BOND_FDIM, HIDDEN), lambda i: (0, 0)),
            pl.BlockSpec((HIDDEN, HIDDEN), lambda i: (0, 0)),
        ],
        out_specs=[
            pl.BlockSpec((BR, HIDDEN), lambda i: (i, 0)),
            pl.BlockSpec((BR, HIDDEN), lambda i: (i, 0)),
        ],
        out_shape=[
            jax.ShapeDtypeStruct((N_BONDS, HIDDEN), jnp.float32),
            jax.ShapeDtypeStruct((N_BONDS, HIDDEN), jnp.float32),
        ],
    )(f_bonds, W_i, W_h)


def _tc_step(premsg, W_h):
    def body(pm_ref, wh_ref, nmh_ref):
        msg = jnp.maximum(pm_ref[...], 0.0)
        nmh_ref[...] = -jnp.dot(msg, wh_ref[...], preferred_element_type=jnp.float32)

    return pl.pallas_call(
        body,
        grid=(N_BONDS // BR,),
        in_specs=[
            pl.BlockSpec((BR, HIDDEN), lambda i: (i, 0)),
            pl.BlockSpec((HIDDEN, HIDDEN), lambda i: (0, 0)),
        ],
        out_specs=pl.BlockSpec((BR, HIDDEN), lambda i: (i, 0)),
        out_shape=jax.ShapeDtypeStruct((N_BONDS, HIDDEN), jnp.float32),
    )(premsg, W_h)


def _tc_readout(f_atoms, amsg, Wo_a, Wo_m, b_o2, seg2d):
    n_blocks = N_ATOMS // DR

    def body(fa_ref, am_ref, seg_ref, woa_ref, wom_ref, bo_ref, out_ref,
             sums_v, cnts_v):
        k = pl.program_id(0)

        @pl.when(k == 0)
        def _():
            sums_v[...] = jnp.zeros_like(sums_v)
            cnts_v[...] = jnp.zeros_like(cnts_v)

        hid = jnp.dot(fa_ref[...], woa_ref[...], preferred_element_type=jnp.float32)
        hid += jnp.dot(am_ref[...], wom_ref[...], preferred_element_type=jnp.float32)
        hid = jnp.maximum(hid + bo_ref[...], 0.0)
        rows = lax.broadcasted_iota(jnp.int32, (N_MOLS, DR), 0)
        oh = (rows == seg_ref[0]).astype(jnp.float32)
        sums_v[...] += jnp.dot(oh, hid, preferred_element_type=jnp.float32)
        cnts_v[...] += jnp.sum(oh, axis=1, keepdims=True)

        @pl.when(k == n_blocks - 1)
        def _():
            out_ref[...] = sums_v[...] / jnp.maximum(cnts_v[...], 1.0)

    return pl.pallas_call(
        body,
        grid=(n_blocks,),
        in_specs=[
            pl.BlockSpec((DR, ATOM_FDIM), lambda k: (k, 0)),
            pl.BlockSpec((DR, HIDDEN), lambda k: (k, 0)),
            pl.BlockSpec((1, 1, DR), lambda k: (k, 0, 0)),
            pl.BlockSpec((ATOM_FDIM, HIDDEN), lambda k: (0, 0)),
            pl.BlockSpec((HIDDEN, HIDDEN), lambda k: (0, 0)),
            pl.BlockSpec((1, HIDDEN), lambda k: (0, 0)),
        ],
        out_specs=pl.BlockSpec((N_MOLS, HIDDEN), lambda k: (0, 0)),
        out_shape=jax.ShapeDtypeStruct((N_MOLS, HIDDEN), jnp.float32),
        scratch_shapes=[
            pltpu.VMEM((N_MOLS, HIDDEN), jnp.float32),
            pltpu.VMEM((N_MOLS, 1), jnp.float32),
        ],
    )(f_atoms, amsg, seg2d, Wo_a, Wo_m, b_o2)


def kernel(f_atoms, f_bonds, W_i, W_h, W_o, b_o, a2b, b2a, b2revb, segment_ids):
    a2b = a2b.astype(jnp.int32)
    b2a = b2a.astype(jnp.int32)
    b2revb = b2revb.astype(jnp.int32)
    segment_ids = segment_ids.astype(jnp.int32)

    n_pad = NA_PAD - N_ATOMS
    pad_block = (
        jnp.arange(n_pad * MAX_NB, dtype=jnp.int32) % N_BONDS
    ).reshape(n_pad, MAX_NB)
    a2bS = (
        jnp.concatenate([a2b, pad_block], axis=0)
        .reshape(NCHUNK_A, CA, MAX_NB)
        .transpose(0, 2, 1)
    )
    Wo_a = W_o[:ATOM_FDIM]
    Wo_m = W_o[ATOM_FDIM:]
    b_o2 = b_o.reshape(1, HIDDEN)
    seg2d = segment_ids.reshape(N_ATOMS // DR, 1, DR)

    gather_neg = _make_gather_sum(negate=True)
    gather_plain = _make_gather_sum(negate=False)
    combine = _make_combine(relu=False)
    combine_relu = _make_combine(relu=True)

    inp, nmh1 = _tc_init(f_bonds, W_i, W_h)
    a1 = gather_neg(nmh1, a2bS)
    premsg2 = combine(inp, a1, nmh1, b2a, b2revb)
    nmh2 = _tc_step(premsg2, W_h)
    a2 = gather_neg(nmh2, a2bS)
    msg3 = combine_relu(inp, a2, nmh2, b2a, b2revb)
    amsg3 = gather_plain(msg3, a2bS)
    return _tc_readout(f_atoms, amsg3[:N_ATOMS], Wo_a, Wo_m, b_o2, seg2d)


# speedup vs baseline: 1.6554x; 1.0726x over previous
import functools

import jax
import jax.numpy as jnp
from jax import lax
from jax.experimental import pallas as pl
from jax.experimental.pallas import tpu as pltpu
from jax.experimental.pallas import tpu_sc as plsc

N_ATOMS = 10000
N_BONDS = 320000
MAX_NB = 32
ATOM_FDIM = 133
BOND_FDIM = 147
HIDDEN = 128
N_MOLS = 500

NC = 2
NS = 16
NW = NC * NS
LANES = 16

NA_PAD = 10240
CA = 64
NCHUNK_A = NA_PAD // CA
NGRP = 4

CB = 256
NCHUNK_B = N_BONDS // CB
KS = CB // 128

BR = 1280
DR = 1000


def _sc_mesh():
    return plsc.VectorSubcoreMesh(
        core_axis_name="c", subcore_axis_name="s", num_cores=NC, num_subcores=NS
    )


def _make_gather_sum(negate: bool):
    def body(table_hbm, a2bS_hbm, out_hbm, idx_v, acc_v, sem0, sem1):
        wid = lax.axis_index("s") * NC + lax.axis_index("c")

        def chunk(ci, carry):
            cid = wid + ci * NW

            @pl.when(cid < NCHUNK_A)
            def _():
                abase = cid * CA
                pltpu.sync_copy(a2bS_hbm.at[cid], idx_v)
                leaders = [
                    pltpu.async_copy(
                        table_hbm.at[idx_v.at[g * 8]],
                        acc_v.at[pl.ds(g * CA, CA)],
                        sem0,
                    )
                    for g in range(NGRP)
                ]
                for d in leaders:
                    d.wait()
                descs = [
                    pltpu.async_copy(
                        table_hbm.at[idx_v.at[g * 8 + 1 + j]],
                        acc_v.at[pl.ds(g * CA, CA)],
                        sem1,
                        add=True,
                    )
                    for g in range(NGRP)
                    for j in range(7)
                ]
                for d in descs:
                    d.wait()

                sgn = -1.0 if negate else 1.0

                def red_row(i, c3):
                    for j in range(8):
                        sl = pl.ds(j * LANES, LANES)
                        v = acc_v[i, sl]
                        for g in range(1, NGRP):
                            v = v + acc_v[g * CA + i, sl]
                        acc_v[i, sl] = sgn * v
                    return c3

                lax.fori_loop(0, CA, red_row, 0)

                pltpu.sync_copy(
                    acc_v.at[pl.ds(0, CA)], out_hbm.at[pl.ds(abase, CA)]
                )

            return carry

        lax.fori_loop(0, (NCHUNK_A + NW - 1) // NW, chunk, 0)

    return pl.kernel(
        body,
        out_type=jax.ShapeDtypeStruct((NA_PAD, HIDDEN), jnp.float32),
        mesh=_sc_mesh(),
        scratch_types=[
            pltpu.VMEM((MAX_NB, CA), jnp.int32),
            pltpu.VMEM((NGRP * CA, HIDDEN), jnp.float32),
            pltpu.SemaphoreType.DMA,
            pltpu.SemaphoreType.DMA,
        ],
    )


def _make_combine(relu: bool):
    def body(inp_hbm, atab_hbm, ntab_hbm, b2a_hbm, b2revb_hbm, out_hbm,
             idxa0, idxr0, buf0, idxa1, idxr1, buf1, psem0, psem1,
             gsem0, gsem1):
        wid = lax.axis_index("s") * NC + lax.axis_index("c")
        bufs = ((idxa0, idxr0, buf0, psem0, gsem0),
                (idxa1, idxr1, buf1, psem1, gsem1))

        def prefetch(cid, bset):
            idxa, idxr, buf, psem, _ = bset

            @pl.when((cid >= 0) & (cid < NCHUNK_B))
            def _():
                base = cid * CB
                pltpu.async_copy(b2a_hbm.at[pl.ds(base, CB)], idxa, psem)
                pltpu.async_copy(b2revb_hbm.at[pl.ds(base, CB)], idxr, psem)
                pltpu.async_copy(inp_hbm.at[pl.ds(base, CB)], buf, psem)

        def fire(cid, bset):
            idxa, idxr, buf, psem, gsem = bset

            @pl.when((cid >= 0) & (cid < NCHUNK_B))
            def _():
                base = cid * CB
                pltpu.make_async_copy(
                    b2a_hbm.at[pl.ds(base, CB)], idxa, psem).wait()
                pltpu.make_async_copy(
                    b2revb_hbm.at[pl.ds(base, CB)], idxr, psem).wait()
                pltpu.make_async_copy(
                    inp_hbm.at[pl.ds(base, CB)], buf, psem).wait()
                for j in range(KS):
                    pltpu.async_copy(
                        atab_hbm.at[idxa.at[pl.ds(j * 128, 128)]],
                        buf.at[pl.ds(j * 128, 128)], gsem, add=True)
                for j in range(KS):
                    pltpu.async_copy(
                        ntab_hbm.at[idxr.at[pl.ds(j * 128, 128)]],
                        buf.at[pl.ds(j * 128, 128)], gsem, add=True)

        def finish(cid, bset):
            idxa, idxr, buf, psem, gsem = bset

            @pl.when((cid >= 0) & (cid < NCHUNK_B))
            def _():
                base = cid * CB
                for _ in range(2 * KS):
                    pltpu.make_async_copy(
                        inp_hbm.at[pl.ds(base, 128)],
                        buf.at[pl.ds(0, 128)], gsem).wait()

                if relu:
                    def relu_row(i, c2):
                        for j in range(8):
                            sl = pl.ds(j * LANES, LANES)
                            buf[i, sl] = jnp.maximum(buf[i, sl], 0.0)
                        return c2

                    lax.fori_loop(0, CB, relu_row, 0)

                pltpu.sync_copy(buf, out_hbm.at[pl.ds(base, CB)])

        prefetch(wid, bufs[0])

        def pair(k, carry):
            c0 = wid + (2 * k) * NW
            c1 = wid + (2 * k + 1) * NW
            fire(c0, bufs[0])
            finish(c1 - 2 * NW, bufs[1])
            prefetch(c1, bufs[1])
            fire(c1, bufs[1])
            finish(c0, bufs[0])
            prefetch(c0 + 2 * NW, bufs[0])
            return carry

        npair = (NCHUNK_B + 2 * NW - 1) // (2 * NW) + 1
        lax.fori_loop(0, npair, pair, 0)

    return pl.kernel(
        body,
        out_type=jax.ShapeDtypeStruct((N_BONDS, HIDDEN), jnp.float32),
        mesh=_sc_mesh(),
        scratch_types=[
            pltpu.VMEM((CB,), jnp.int32),
            pltpu.VMEM((CB,), jnp.int32),
            pltpu.VMEM((CB, HIDDEN), jnp.float32),
            pltpu.VMEM((CB,), jnp.int32),
            pltpu.VMEM((CB,), jnp.int32),
            pltpu.VMEM((CB, HIDDEN), jnp.float32),
            pltpu.SemaphoreType.DMA,
            pltpu.SemaphoreType.DMA,
            pltpu.SemaphoreType.DMA,
            pltpu.SemaphoreType.DMA,
        ],
    )


def _tc_init(f_bonds, W_i, W_h):
    def body(fb_ref, wi_ref, wh_ref, inp_ref, nmh_ref):
        inp = jnp.dot(fb_ref[...], wi_ref[...], preferred_element_type=jnp.float32)
        inp_ref[...] = inp
        msg = jnp.maximum(inp, 0.0)
        nmh_ref[...] = -jnp.dot(msg, wh_ref[...], preferred_element_type=jnp.float32)

    return pl.pallas_call(
        body,
        grid=(N_BONDS // BR,),
        in_specs=[
            pl.BlockSpec((BR, BOND_FDIM), lambda i: (i, 0)),
            pl.BlockSpec((BOND_FDIM, HIDDEN), lambda i: (0, 0)),
            pl.BlockSpec((HIDDEN, HIDDEN), lambda i: (0, 0)),
        ],
        out_specs=[
            pl.BlockSpec((BR, HIDDEN), lambda i: (i, 0)),
            pl.BlockSpec((BR, HIDDEN), lambda i: (i, 0)),
        ],
        out_shape=[
            jax.ShapeDtypeStruct((N_BONDS, HIDDEN), jnp.float32),
            jax.ShapeDtypeStruct((N_BONDS, HIDDEN), jnp.float32),
        ],
    )(f_bonds, W_i, W_h)


def _tc_step(premsg, W_h):
    def body(pm_ref, wh_ref, nmh_ref):
        msg = jnp.maximum(pm_ref[...], 0.0)
        nmh_ref[...] = -jnp.dot(msg, wh_ref[...], preferred_element_type=jnp.float32)

    return pl.pallas_call(
        body,
        grid=(N_BONDS // BR,),
        in_specs=[
            pl.BlockSpec((BR, HIDDEN), lambda i: (i, 0)),
            pl.BlockSpec((HIDDEN, HIDDEN), lambda i: (0, 0)),
        ],
        out_specs=pl.BlockSpec((BR, HIDDEN), lambda i: (i, 0)),
        out_shape=jax.ShapeDtypeStruct((N_BONDS, HIDDEN), jnp.float32),
    )(premsg, W_h)


def _tc_readout(f_atoms, amsg, Wo_a, Wo_m, b_o2, seg2d):
    n_blocks = N_ATOMS // DR

    def body(fa_ref, am_ref, seg_ref, woa_ref, wom_ref, bo_ref, out_ref,
             sums_v, cnts_v):
        k = pl.program_id(0)

        @pl.when(k == 0)
        def _():
            sums_v[...] = jnp.zeros_like(sums_v)
            cnts_v[...] = jnp.zeros_like(cnts_v)

        hid = jnp.dot(fa_ref[...], woa_ref[...], preferred_element_type=jnp.float32)
        hid += jnp.dot(am_ref[...], wom_ref[...], preferred_element_type=jnp.float32)
        hid = jnp.maximum(hid + bo_ref[...], 0.0)
        rows = lax.broadcasted_iota(jnp.int32, (N_MOLS, DR), 0)
        oh = (rows == seg_ref[0]).astype(jnp.float32)
        sums_v[...] += jnp.dot(oh, hid, preferred_element_type=jnp.float32)
        cnts_v[...] += jnp.sum(oh, axis=1, keepdims=True)

        @pl.when(k == n_blocks - 1)
        def _():
            out_ref[...] = sums_v[...] / jnp.maximum(cnts_v[...], 1.0)

    return pl.pallas_call(
        body,
        grid=(n_blocks,),
        in_specs=[
            pl.BlockSpec((DR, ATOM_FDIM), lambda k: (k, 0)),
            pl.BlockSpec((DR, HIDDEN), lambda k: (k, 0)),
            pl.BlockSpec((1, 1, DR), lambda k: (k, 0, 0)),
            pl.BlockSpec((ATOM_FDIM, HIDDEN), lambda k: (0, 0)),
            pl.BlockSpec((HIDDEN, HIDDEN), lambda k: (0, 0)),
            pl.BlockSpec((1, HIDDEN), lambda k: (0, 0)),
        ],
        out_specs=pl.BlockSpec((N_MOLS, HIDDEN), lambda k: (0, 0)),
        out_shape=jax.ShapeDtypeStruct((N_MOLS, HIDDEN), jnp.float32),
        scratch_shapes=[
            pltpu.VMEM((N_MOLS, HIDDEN), jnp.float32),
            pltpu.VMEM((N_MOLS, 1), jnp.float32),
        ],
    )(f_atoms, amsg, seg2d, Wo_a, Wo_m, b_o2)


def kernel(f_atoms, f_bonds, W_i, W_h, W_o, b_o, a2b, b2a, b2revb, segment_ids):
    a2b = a2b.astype(jnp.int32)
    b2a = b2a.astype(jnp.int32)
    b2revb = b2revb.astype(jnp.int32)
    segment_ids = segment_ids.astype(jnp.int32)

    n_pad = NA_PAD - N_ATOMS
    pad_block = (
        jnp.arange(n_pad * MAX_NB, dtype=jnp.int32) % N_BONDS
    ).reshape(n_pad, MAX_NB)
    a2bS = (
        jnp.concatenate([a2b, pad_block], axis=0)
        .reshape(NCHUNK_A, CA, MAX_NB)
        .transpose(0, 2, 1)
    )
    Wo_a = W_o[:ATOM_FDIM]
    Wo_m = W_o[ATOM_FDIM:]
    b_o2 = b_o.reshape(1, HIDDEN)
    seg2d = segment_ids.reshape(N_ATOMS // DR, 1, DR)

    gather_neg = _make_gather_sum(negate=True)
    gather_plain = _make_gather_sum(negate=False)
    combine = _make_combine(relu=False)
    combine_relu = _make_combine(relu=True)

    inp, nmh1 = _tc_init(f_bonds, W_i, W_h)
    a1 = gather_neg(nmh1, a2bS)
    premsg2 = combine(inp, a1, nmh1, b2a, b2revb)
    nmh2 = _tc_step(premsg2, W_h)
    a2 = gather_neg(nmh2, a2bS)
    msg3 = combine_relu(inp, a2, nmh2, b2a, b2revb)
    amsg3 = gather_plain(msg3, a2bS)
    return _tc_readout(f_atoms, amsg3[:N_ATOMS], Wo_a, Wo_m, b_o2, seg2d)


# speedup vs baseline: 1.8362x; 1.1092x over previous
import functools

import jax
import jax.numpy as jnp
from jax import lax
from jax.experimental import pallas as pl
from jax.experimental.pallas import tpu as pltpu
from jax.experimental.pallas import tpu_sc as plsc

N_ATOMS = 10000
N_BONDS = 320000
MAX_NB = 32
ATOM_FDIM = 133
BOND_FDIM = 147
HIDDEN = 128
N_MOLS = 500

NC = 2
NS = 16
NW = NC * NS
LANES = 16

NA_PAD = 10240
CA = 64
NCHUNK_A = NA_PAD // CA
NGRP = 4

CB = 256
NCHUNK_B = N_BONDS // CB
KS = CB // 128

BR = 2560
DR = 1000


def _sc_mesh():
    return plsc.VectorSubcoreMesh(
        core_axis_name="c", subcore_axis_name="s", num_cores=NC, num_subcores=NS
    )


def _make_gather_sum(negate: bool):
    def body(table_hbm, a2bS_hbm, out_hbm, idx_v, acc_v, sem0, sem1):
        wid = lax.axis_index("s") * NC + lax.axis_index("c")

        def chunk(ci, carry):
            cid = wid + ci * NW

            @pl.when(cid < NCHUNK_A)
            def _():
                abase = cid * CA
                pltpu.sync_copy(a2bS_hbm.at[cid], idx_v)
                leaders = [
                    pltpu.async_copy(
                        table_hbm.at[idx_v.at[g * 8]],
                        acc_v.at[pl.ds(g * CA, CA)],
                        sem0,
                    )
                    for g in range(NGRP)
                ]
                for d in leaders:
                    d.wait()
                descs = [
                    pltpu.async_copy(
                        table_hbm.at[idx_v.at[g * 8 + 1 + j]],
                        acc_v.at[pl.ds(g * CA, CA)],
                        sem1,
                        add=True,
                    )
                    for g in range(NGRP)
                    for j in range(7)
                ]
                for d in descs:
                    d.wait()

                sgn = -1.0 if negate else 1.0

                def red_row(i, c3):
                    for j in range(8):
                        sl = pl.ds(j * LANES, LANES)
                        v = acc_v[i, sl]
                        for g in range(1, NGRP):
                            v = v + acc_v[g * CA + i, sl]
                        acc_v[i, sl] = sgn * v
                    return c3

                lax.fori_loop(0, CA, red_row, 0)

                pltpu.sync_copy(
                    acc_v.at[pl.ds(0, CA)], out_hbm.at[pl.ds(abase, CA)]
                )

            return carry

        lax.fori_loop(0, (NCHUNK_A + NW - 1) // NW, chunk, 0)

    return pl.kernel(
        body,
        out_type=jax.ShapeDtypeStruct((NA_PAD, HIDDEN), jnp.float32),
        mesh=_sc_mesh(),
        scratch_types=[
            pltpu.VMEM((MAX_NB, CA), jnp.int32),
            pltpu.VMEM((NGRP * CA, HIDDEN), jnp.float32),
            pltpu.SemaphoreType.DMA,
            pltpu.SemaphoreType.DMA,
        ],
    )


def _make_combine(relu: bool):
    def body(inp_hbm, atab_hbm, ntab_hbm, b2a_hbm, b2revb_hbm, out_hbm,
             idxa0, idxr0, buf0, idxa1, idxr1, buf1, psem0, psem1,
             gsem0, gsem1):
        wid = lax.axis_index("s") * NC + lax.axis_index("c")
        bufs = ((idxa0, idxr0, buf0, psem0, gsem0),
                (idxa1, idxr1, buf1, psem1, gsem1))

        def prefetch(cid, bset):
            idxa, idxr, buf, psem, _ = bset

            @pl.when((cid >= 0) & (cid < NCHUNK_B))
            def _():
                base = cid * CB
                pltpu.async_copy(b2a_hbm.at[pl.ds(base, CB)], idxa, psem)
                pltpu.async_copy(b2revb_hbm.at[pl.ds(base, CB)], idxr, psem)
                pltpu.async_copy(inp_hbm.at[pl.ds(base, CB)], buf, psem)

        def fire(cid, bset):
            idxa, idxr, buf, psem, gsem = bset

            @pl.when((cid >= 0) & (cid < NCHUNK_B))
            def _():
                base = cid * CB
                pltpu.make_async_copy(
                    b2a_hbm.at[pl.ds(base, CB)], idxa, psem).wait()
                pltpu.make_async_copy(
                    b2revb_hbm.at[pl.ds(base, CB)], idxr, psem).wait()
                pltpu.make_async_copy(
                    inp_hbm.at[pl.ds(base, CB)], buf, psem).wait()
                for j in range(KS):
                    pltpu.async_copy(
                        atab_hbm.at[idxa.at[pl.ds(j * 128, 128)]],
                        buf.at[pl.ds(j * 128, 128)], gsem, add=True)
                for j in range(KS):
                    pltpu.async_copy(
                        ntab_hbm.at[idxr.at[pl.ds(j * 128, 128)]],
                        buf.at[pl.ds(j * 128, 128)], gsem, add=True)

        def finish(cid, bset):
            idxa, idxr, buf, psem, gsem = bset

            @pl.when((cid >= 0) & (cid < NCHUNK_B))
            def _():
                base = cid * CB
                for _ in range(2 * KS):
                    pltpu.make_async_copy(
                        inp_hbm.at[pl.ds(base, 128)],
                        buf.at[pl.ds(0, 128)], gsem).wait()

                if relu:
                    def relu_row(i, c2):
                        for j in range(8):
                            sl = pl.ds(j * LANES, LANES)
                            buf[i, sl] = jnp.maximum(buf[i, sl], 0.0)
                        return c2

                    lax.fori_loop(0, CB, relu_row, 0)

                pltpu.sync_copy(buf, out_hbm.at[pl.ds(base, CB)])

        prefetch(wid, bufs[0])

        def pair(k, carry):
            c0 = wid + (2 * k) * NW
            c1 = wid + (2 * k + 1) * NW
            fire(c0, bufs[0])
            finish(c1 - 2 * NW, bufs[1])
            prefetch(c1, bufs[1])
            fire(c1, bufs[1])
            finish(c0, bufs[0])
            prefetch(c0 + 2 * NW, bufs[0])
            return carry

        npair = (NCHUNK_B + 2 * NW - 1) // (2 * NW) + 1
        lax.fori_loop(0, npair, pair, 0)

    return pl.kernel(
        body,
        out_type=jax.ShapeDtypeStruct((N_BONDS, HIDDEN), jnp.float32),
        mesh=_sc_mesh(),
        scratch_types=[
            pltpu.VMEM((CB,), jnp.int32),
            pltpu.VMEM((CB,), jnp.int32),
            pltpu.VMEM((CB, HIDDEN), jnp.float32),
            pltpu.VMEM((CB,), jnp.int32),
            pltpu.VMEM((CB,), jnp.int32),
            pltpu.VMEM((CB, HIDDEN), jnp.float32),
            pltpu.SemaphoreType.DMA,
            pltpu.SemaphoreType.DMA,
            pltpu.SemaphoreType.DMA,
            pltpu.SemaphoreType.DMA,
        ],
    )


def _tc_init(f_bonds, W_i, W_h):
    def body(fb_ref, wi_ref, wh_ref, inp_ref, nmh_ref):
        inp = jnp.dot(fb_ref[...], wi_ref[...], preferred_element_type=jnp.float32)
        inp_ref[...] = inp
        msg = jnp.maximum(inp, 0.0)
        nmh_ref[...] = -jnp.dot(msg, wh_ref[...], preferred_element_type=jnp.float32)

    return pl.pallas_call(
        body,
        grid=(N_BONDS // BR,),
        in_specs=[
            pl.BlockSpec((BR, BOND_FDIM), lambda i: (i, 0)),
            pl.BlockSpec((BOND_FDIM, HIDDEN), lambda i: (0, 0)),
            pl.BlockSpec((HIDDEN, HIDDEN), lambda i: (0, 0)),
        ],
        out_specs=[
            pl.BlockSpec((BR, HIDDEN), lambda i: (i, 0)),
            pl.BlockSpec((BR, HIDDEN), lambda i: (i, 0)),
        ],
        out_shape=[
            jax.ShapeDtypeStruct((N_BONDS, HIDDEN), jnp.float32),
            jax.ShapeDtypeStruct((N_BONDS, HIDDEN), jnp.float32),
        ],
    )(f_bonds, W_i, W_h)


def _tc_step(premsg, W_h):
    def body(pm_ref, wh_ref, nmh_ref):
        msg = jnp.maximum(pm_ref[...], 0.0)
        nmh_ref[...] = -jnp.dot(msg, wh_ref[...], preferred_element_type=jnp.float32)

    return pl.pallas_call(
        body,
        grid=(N_BONDS // BR,),
        in_specs=[
            pl.BlockSpec((BR, HIDDEN), lambda i: (i, 0)),
            pl.BlockSpec((HIDDEN, HIDDEN), lambda i: (0, 0)),
        ],
        out_specs=pl.BlockSpec((BR, HIDDEN), lambda i: (i, 0)),
        out_shape=jax.ShapeDtypeStruct((N_BONDS, HIDDEN), jnp.float32),
    )(premsg, W_h)


def _tc_readout(f_atoms, amsg, Wo_a, Wo_m, b_o2, seg2d):
    n_blocks = N_ATOMS // DR

    def body(fa_ref, am_ref, seg_ref, woa_ref, wom_ref, bo_ref, out_ref,
             sums_v, cnts_v):
        k = pl.program_id(0)

        @pl.when(k == 0)
        def _():
            sums_v[...] = jnp.zeros_like(sums_v)
            cnts_v[...] = jnp.zeros_like(cnts_v)

        hid = jnp.dot(fa_ref[...], woa_ref[...], preferred_element_type=jnp.float32)
        hid += jnp.dot(am_ref[...], wom_ref[...], preferred_element_type=jnp.float32)
        hid = jnp.maximum(hid + bo_ref[...], 0.0)
        rows = lax.broadcasted_iota(jnp.int32, (N_MOLS, DR), 0)
        oh = (rows == seg_ref[0]).astype(jnp.float32)
        sums_v[...] += jnp.dot(oh, hid, preferred_element_type=jnp.float32)
        cnts_v[...] += jnp.sum(oh, axis=1, keepdims=True)

        @pl.when(k == n_blocks - 1)
        def _():
            out_ref[...] = sums_v[...] / jnp.maximum(cnts_v[...], 1.0)

    return pl.pallas_call(
        body,
        grid=(n_blocks,),
        in_specs=[
            pl.BlockSpec((DR, ATOM_FDIM), lambda k: (k, 0)),
            pl.BlockSpec((DR, HIDDEN), lambda k: (k, 0)),
            pl.BlockSpec((1, 1, DR), lambda k: (k, 0, 0)),
            pl.BlockSpec((ATOM_FDIM, HIDDEN), lambda k: (0, 0)),
            pl.BlockSpec((HIDDEN, HIDDEN), lambda k: (0, 0)),
            pl.BlockSpec((1, HIDDEN), lambda k: (0, 0)),
        ],
        out_specs=pl.BlockSpec((N_MOLS, HIDDEN), lambda k: (0, 0)),
        out_shape=jax.ShapeDtypeStruct((N_MOLS, HIDDEN), jnp.float32),
        scratch_shapes=[
            pltpu.VMEM((N_MOLS, HIDDEN), jnp.float32),
            pltpu.VMEM((N_MOLS, 1), jnp.float32),
        ],
    )(f_atoms, amsg, seg2d, Wo_a, Wo_m, b_o2)


def kernel(f_atoms, f_bonds, W_i, W_h, W_o, b_o, a2b, b2a, b2revb, segment_ids):
    a2b = a2b.astype(jnp.int32)
    b2a = b2a.astype(jnp.int32)
    b2revb = b2revb.astype(jnp.int32)
    segment_ids = segment_ids.astype(jnp.int32)

    n_pad = NA_PAD - N_ATOMS
    pad_block = (
        jnp.arange(n_pad * MAX_NB, dtype=jnp.int32) % N_BONDS
    ).reshape(n_pad, MAX_NB)
    a2bS = (
        jnp.concatenate([a2b, pad_block], axis=0)
        .reshape(NCHUNK_A, CA, MAX_NB)
        .transpose(0, 2, 1)
    )
    Wo_a = W_o[:ATOM_FDIM]
    Wo_m = W_o[ATOM_FDIM:]
    b_o2 = b_o.reshape(1, HIDDEN)
    seg2d = segment_ids.reshape(N_ATOMS // DR, 1, DR)

    gather_neg = _make_gather_sum(negate=True)
    gather_plain = _make_gather_sum(negate=False)
    combine = _make_combine(relu=False)
    combine_relu = _make_combine(relu=True)

    inp, nmh1 = _tc_init(f_bonds, W_i, W_h)
    a1 = gather_neg(nmh1, a2bS)
    premsg2 = combine(inp, a1, nmh1, b2a, b2revb)
    nmh2 = _tc_step(premsg2, W_h)
    a2 = gather_neg(nmh2, a2bS)
    msg3 = combine_relu(inp, a2, nmh2, b2a, b2revb)
    amsg3 = gather_plain(msg3, a2bS)
    return _tc_readout(f_atoms, amsg3[:N_ATOMS], Wo_a, Wo_m, b_o2, seg2d)


# speedup vs baseline: 1.9447x; 1.0591x over previous
import functools

import jax
import jax.numpy as jnp
from jax import lax
from jax.experimental import pallas as pl
from jax.experimental.pallas import tpu as pltpu
from jax.experimental.pallas import tpu_sc as plsc

N_ATOMS = 10000
N_BONDS = 320000
MAX_NB = 32
ATOM_FDIM = 133
BOND_FDIM = 147
HIDDEN = 128
N_MOLS = 500

NC = 2
NS = 16
NW = NC * NS
LANES = 16

NA_PAD = 10240
CA = 64
NCHUNK_A = NA_PAD // CA
NGRP = 4

CB = 256
NCHUNK_B = N_BONDS // CB
KS = CB // 128

BR = 6400
DR = 1000


def _sc_mesh():
    return plsc.VectorSubcoreMesh(
        core_axis_name="c", subcore_axis_name="s", num_cores=NC, num_subcores=NS
    )


def _make_gather_sum(negate: bool):
    def body(table_hbm, a2bS_hbm, out_hbm, idx_v, acc_v, sem0, sem1):
        wid = lax.axis_index("s") * NC + lax.axis_index("c")

        def chunk(ci, carry):
            cid = wid + ci * NW

            @pl.when(cid < NCHUNK_A)
            def _():
                abase = cid * CA
                pltpu.sync_copy(a2bS_hbm.at[cid], idx_v)
                leaders = [
                    pltpu.async_copy(
                        table_hbm.at[idx_v.at[g * 8]],
                        acc_v.at[pl.ds(g * CA, CA)],
                        sem0,
                    )
                    for g in range(NGRP)
                ]
                for d in leaders:
                    d.wait()
                descs = [
                    pltpu.async_copy(
                        table_hbm.at[idx_v.at[g * 8 + 1 + j]],
                        acc_v.at[pl.ds(g * CA, CA)],
                        sem1,
                        add=True,
                    )
                    for g in range(NGRP)
                    for j in range(7)
                ]
                for d in descs:
                    d.wait()

                sgn = -1.0 if negate else 1.0

                def red_row(i, c3):
                    for j in range(8):
                        sl = pl.ds(j * LANES, LANES)
                        v = acc_v[i, sl]
                        for g in range(1, NGRP):
                            v = v + acc_v[g * CA + i, sl]
                        acc_v[i, sl] = sgn * v
                    return c3

                lax.fori_loop(0, CA, red_row, 0)

                pltpu.sync_copy(
                    acc_v.at[pl.ds(0, CA)], out_hbm.at[pl.ds(abase, CA)]
                )

            return carry

        lax.fori_loop(0, (NCHUNK_A + NW - 1) // NW, chunk, 0)

    return pl.kernel(
        body,
        out_type=jax.ShapeDtypeStruct((NA_PAD, HIDDEN), jnp.float32),
        mesh=_sc_mesh(),
        scratch_types=[
            pltpu.VMEM((MAX_NB, CA), jnp.int32),
            pltpu.VMEM((NGRP * CA, HIDDEN), jnp.float32),
            pltpu.SemaphoreType.DMA,
            pltpu.SemaphoreType.DMA,
        ],
    )


def _make_combine(relu: bool):
    def body(inp_hbm, atab_hbm, ntab_hbm, b2a_hbm, b2revb_hbm, out_hbm,
             idxa0, idxr0, buf0, idxa1, idxr1, buf1, psem0, psem1,
             gsem0, gsem1):
        wid = lax.axis_index("s") * NC + lax.axis_index("c")
        bufs = ((idxa0, idxr0, buf0, psem0, gsem0),
                (idxa1, idxr1, buf1, psem1, gsem1))

        def prefetch(cid, bset):
            idxa, idxr, buf, psem, _ = bset

            @pl.when((cid >= 0) & (cid < NCHUNK_B))
            def _():
                base = cid * CB
                pltpu.async_copy(b2a_hbm.at[pl.ds(base, CB)], idxa, psem)
                pltpu.async_copy(b2revb_hbm.at[pl.ds(base, CB)], idxr, psem)
                pltpu.async_copy(inp_hbm.at[pl.ds(base, CB)], buf, psem)

        def fire(cid, bset):
            idxa, idxr, buf, psem, gsem = bset

            @pl.when((cid >= 0) & (cid < NCHUNK_B))
            def _():
                base = cid * CB
                pltpu.make_async_copy(
                    b2a_hbm.at[pl.ds(base, CB)], idxa, psem).wait()
                pltpu.make_async_copy(
                    b2revb_hbm.at[pl.ds(base, CB)], idxr, psem).wait()
                pltpu.make_async_copy(
                    inp_hbm.at[pl.ds(base, CB)], buf, psem).wait()
                for j in range(KS):
                    pltpu.async_copy(
                        atab_hbm.at[idxa.at[pl.ds(j * 128, 128)]],
                        buf.at[pl.ds(j * 128, 128)], gsem, add=True)
                for j in range(KS):
                    pltpu.async_copy(
                        ntab_hbm.at[idxr.at[pl.ds(j * 128, 128)]],
                        buf.at[pl.ds(j * 128, 128)], gsem, add=True)

        def finish(cid, bset):
            idxa, idxr, buf, psem, gsem = bset

            @pl.when((cid >= 0) & (cid < NCHUNK_B))
            def _():
                base = cid * CB
                for _ in range(2 * KS):
                    pltpu.make_async_copy(
                        inp_hbm.at[pl.ds(base, 128)],
                        buf.at[pl.ds(0, 128)], gsem).wait()

                if relu:
                    def relu_row(i, c2):
                        for j in range(8):
                            sl = pl.ds(j * LANES, LANES)
                            buf[i, sl] = jnp.maximum(buf[i, sl], 0.0)
                        return c2

                    lax.fori_loop(0, CB, relu_row, 0)

                pltpu.sync_copy(buf, out_hbm.at[pl.ds(base, CB)])

        prefetch(wid, bufs[0])

        def pair(k, carry):
            c0 = wid + (2 * k) * NW
            c1 = wid + (2 * k + 1) * NW
            fire(c0, bufs[0])
            finish(c1 - 2 * NW, bufs[1])
            prefetch(c1, bufs[1])
            fire(c1, bufs[1])
            finish(c0, bufs[0])
            prefetch(c0 + 2 * NW, bufs[0])
            return carry

        npair = (NCHUNK_B + 2 * NW - 1) // (2 * NW) + 1
        lax.fori_loop(0, npair, pair, 0)

    return pl.kernel(
        body,
        out_type=jax.ShapeDtypeStruct((N_BONDS, HIDDEN), jnp.float32),
        mesh=_sc_mesh(),
        scratch_types=[
            pltpu.VMEM((CB,), jnp.int32),
            pltpu.VMEM((CB,), jnp.int32),
            pltpu.VMEM((CB, HIDDEN), jnp.float32),
            pltpu.VMEM((CB,), jnp.int32),
            pltpu.VMEM((CB,), jnp.int32),
            pltpu.VMEM((CB, HIDDEN), jnp.float32),
            pltpu.SemaphoreType.DMA,
            pltpu.SemaphoreType.DMA,
            pltpu.SemaphoreType.DMA,
            pltpu.SemaphoreType.DMA,
        ],
    )


def _tc_init(f_bonds, W_i, W_h):
    def body(fb_ref, wi_ref, wh_ref, inp_ref, nmh_ref):
        inp = jnp.dot(fb_ref[...], wi_ref[...], preferred_element_type=jnp.float32)
        inp_ref[...] = inp
        msg = jnp.maximum(inp, 0.0)
        nmh_ref[...] = -jnp.dot(msg, wh_ref[...], preferred_element_type=jnp.float32)

    return pl.pallas_call(
        body,
        grid=(N_BONDS // BR,),
        in_specs=[
            pl.BlockSpec((BR, BOND_FDIM), lambda i: (i, 0)),
            pl.BlockSpec((BOND_FDIM, HIDDEN), lambda i: (0, 0)),
            pl.BlockSpec((HIDDEN, HIDDEN), lambda i: (0, 0)),
        ],
        out_specs=[
            pl.BlockSpec((BR, HIDDEN), lambda i: (i, 0)),
            pl.BlockSpec((BR, HIDDEN), lambda i: (i, 0)),
        ],
        out_shape=[
            jax.ShapeDtypeStruct((N_BONDS, HIDDEN), jnp.float32),
            jax.ShapeDtypeStruct((N_BONDS, HIDDEN), jnp.float32),
        ],
    )(f_bonds, W_i, W_h)


def _tc_step(premsg, W_h):
    def body(pm_ref, wh_ref, nmh_ref):
        msg = jnp.maximum(pm_ref[...], 0.0)
        nmh_ref[...] = -jnp.dot(msg, wh_ref[...], preferred_element_type=jnp.float32)

    return pl.pallas_call(
        body,
        grid=(N_BONDS // BR,),
        in_specs=[
            pl.BlockSpec((BR, HIDDEN), lambda i: (i, 0)),
            pl.BlockSpec((HIDDEN, HIDDEN), lambda i: (0, 0)),
        ],
        out_specs=pl.BlockSpec((BR, HIDDEN), lambda i: (i, 0)),
        out_shape=jax.ShapeDtypeStruct((N_BONDS, HIDDEN), jnp.float32),
    )(premsg, W_h)


def _tc_readout(f_atoms, amsg, Wo_a, Wo_m, b_o2, seg2d):
    n_blocks = N_ATOMS // DR

    def body(fa_ref, am_ref, seg_ref, woa_ref, wom_ref, bo_ref, out_ref,
             sums_v, cnts_v):
        k = pl.program_id(0)

        @pl.when(k == 0)
        def _():
            sums_v[...] = jnp.zeros_like(sums_v)
            cnts_v[...] = jnp.zeros_like(cnts_v)

        hid = jnp.dot(fa_ref[...], woa_ref[...], preferred_element_type=jnp.float32)
        hid += jnp.dot(am_ref[...], wom_ref[...], preferred_element_type=jnp.float32)
        hid = jnp.maximum(hid + bo_ref[...], 0.0)
        rows = lax.broadcasted_iota(jnp.int32, (N_MOLS, DR), 0)
        oh = (rows == seg_ref[0]).astype(jnp.float32)
        sums_v[...] += jnp.dot(oh, hid, preferred_element_type=jnp.float32)
        cnts_v[...] += jnp.sum(oh, axis=1, keepdims=True)

        @pl.when(k == n_blocks - 1)
        def _():
            out_ref[...] = sums_v[...] / jnp.maximum(cnts_v[...], 1.0)

    return pl.pallas_call(
        body,
        grid=(n_blocks,),
        in_specs=[
            pl.BlockSpec((DR, ATOM_FDIM), lambda k: (k, 0)),
            pl.BlockSpec((DR, HIDDEN), lambda k: (k, 0)),
            pl.BlockSpec((1, 1, DR), lambda k: (k, 0, 0)),
            pl.BlockSpec((ATOM_FDIM, HIDDEN), lambda k: (0, 0)),
            pl.BlockSpec((HIDDEN, HIDDEN), lambda k: (0, 0)),
            pl.BlockSpec((1, HIDDEN), lambda k: (0, 0)),
        ],
        out_specs=pl.BlockSpec((N_MOLS, HIDDEN), lambda k: (0, 0)),
        out_shape=jax.ShapeDtypeStruct((N_MOLS, HIDDEN), jnp.float32),
        scratch_shapes=[
            pltpu.VMEM((N_MOLS, HIDDEN), jnp.float32),
            pltpu.VMEM((N_MOLS, 1), jnp.float32),
        ],
    )(f_atoms, amsg, seg2d, Wo_a, Wo_m, b_o2)


def kernel(f_atoms, f_bonds, W_i, W_h, W_o, b_o, a2b, b2a, b2revb, segment_ids):
    a2b = a2b.astype(jnp.int32)
    b2a = b2a.astype(jnp.int32)
    b2revb = b2revb.astype(jnp.int32)
    segment_ids = segment_ids.astype(jnp.int32)

    n_pad = NA_PAD - N_ATOMS
    pad_block = (
        jnp.arange(n_pad * MAX_NB, dtype=jnp.int32) % N_BONDS
    ).reshape(n_pad, MAX_NB)
    a2bS = (
        jnp.concatenate([a2b, pad_block], axis=0)
        .reshape(NCHUNK_A, CA, MAX_NB)
        .transpose(0, 2, 1)
    )
    Wo_a = W_o[:ATOM_FDIM]
    Wo_m = W_o[ATOM_FDIM:]
    b_o2 = b_o.reshape(1, HIDDEN)
    seg2d = segment_ids.reshape(N_ATOMS // DR, 1, DR)

    gather_neg = _make_gather_sum(negate=True)
    gather_plain = _make_gather_sum(negate=False)
    combine = _make_combine(relu=False)
    combine_relu = _make_combine(relu=True)

    inp, nmh1 = _tc_init(f_bonds, W_i, W_h)
    a1 = gather_neg(nmh1, a2bS)
    premsg2 = combine(inp, a1, nmh1, b2a, b2revb)
    nmh2 = _tc_step(premsg2, W_h)
    a2 = gather_neg(nmh2, a2bS)
    msg3 = combine_relu(inp, a2, nmh2, b2a, b2revb)
    amsg3 = gather_plain(msg3, a2bS)
    return _tc_readout(f_atoms, amsg3[:N_ATOMS], Wo_a, Wo_m, b_o2, seg2d)


# speedup vs baseline: 1.9461x; 1.0007x over previous
import functools

import jax
import jax.numpy as jnp
from jax import lax
from jax.experimental import pallas as pl
from jax.experimental.pallas import tpu as pltpu
from jax.experimental.pallas import tpu_sc as plsc

N_ATOMS = 10000
N_BONDS = 320000
MAX_NB = 32
ATOM_FDIM = 133
BOND_FDIM = 147
HIDDEN = 128
N_MOLS = 500

NC = 2
NS = 16
NW = NC * NS
LANES = 16

NA_PAD = 10240
CA = 64
NCHUNK_A = NA_PAD // CA
NGRP = 4

CB = 256
NCHUNK_B = N_BONDS // CB
KS = CB // 128

BR = 10000
DR = 1000


def _sc_mesh():
    return plsc.VectorSubcoreMesh(
        core_axis_name="c", subcore_axis_name="s", num_cores=NC, num_subcores=NS
    )


def _make_gather_sum(negate: bool):
    def body(table_hbm, a2bS_hbm, out_hbm, idx_v, acc_v, sem0, sem1):
        wid = lax.axis_index("s") * NC + lax.axis_index("c")

        def chunk(ci, carry):
            cid = wid + ci * NW

            @pl.when(cid < NCHUNK_A)
            def _():
                abase = cid * CA
                pltpu.sync_copy(a2bS_hbm.at[cid], idx_v)
                leaders = [
                    pltpu.async_copy(
                        table_hbm.at[idx_v.at[g * 8]],
                        acc_v.at[pl.ds(g * CA, CA)],
                        sem0,
                    )
                    for g in range(NGRP)
                ]
                for d in leaders:
                    d.wait()
                descs = [
                    pltpu.async_copy(
                        table_hbm.at[idx_v.at[g * 8 + 1 + j]],
                        acc_v.at[pl.ds(g * CA, CA)],
                        sem1,
                        add=True,
                    )
                    for g in range(NGRP)
                    for j in range(7)
                ]
                for d in descs:
                    d.wait()

                sgn = -1.0 if negate else 1.0

                def red_row(i, c3):
                    for j in range(8):
                        sl = pl.ds(j * LANES, LANES)
                        v = acc_v[i, sl]
                        for g in range(1, NGRP):
                            v = v + acc_v[g * CA + i, sl]
                        acc_v[i, sl] = sgn * v
                    return c3

                lax.fori_loop(0, CA, red_row, 0)

                pltpu.sync_copy(
                    acc_v.at[pl.ds(0, CA)], out_hbm.at[pl.ds(abase, CA)]
                )

            return carry

        lax.fori_loop(0, (NCHUNK_A + NW - 1) // NW, chunk, 0)

    return pl.kernel(
        body,
        out_type=jax.ShapeDtypeStruct((NA_PAD, HIDDEN), jnp.float32),
        mesh=_sc_mesh(),
        scratch_types=[
            pltpu.VMEM((MAX_NB, CA), jnp.int32),
            pltpu.VMEM((NGRP * CA, HIDDEN), jnp.float32),
            pltpu.SemaphoreType.DMA,
            pltpu.SemaphoreType.DMA,
        ],
    )


def _make_combine(relu: bool):
    def body(inp_hbm, atab_hbm, ntab_hbm, b2a_hbm, b2revb_hbm, out_hbm,
             idxa0, idxr0, buf0, idxa1, idxr1, buf1, psem0, psem1,
             gsem0, gsem1):
        wid = lax.axis_index("s") * NC + lax.axis_index("c")
        bufs = ((idxa0, idxr0, buf0, psem0, gsem0),
                (idxa1, idxr1, buf1, psem1, gsem1))

        def prefetch(cid, bset):
            idxa, idxr, buf, psem, _ = bset

            @pl.when((cid >= 0) & (cid < NCHUNK_B))
            def _():
                base = cid * CB
                pltpu.async_copy(b2a_hbm.at[pl.ds(base, CB)], idxa, psem)
                pltpu.async_copy(b2revb_hbm.at[pl.ds(base, CB)], idxr, psem)
                pltpu.async_copy(inp_hbm.at[pl.ds(base, CB)], buf, psem)

        def fire(cid, bset):
            idxa, idxr, buf, psem, gsem = bset

            @pl.when((cid >= 0) & (cid < NCHUNK_B))
            def _():
                base = cid * CB
                pltpu.make_async_copy(
                    b2a_hbm.at[pl.ds(base, CB)], idxa, psem).wait()
                pltpu.make_async_copy(
                    b2revb_hbm.at[pl.ds(base, CB)], idxr, psem).wait()
                pltpu.make_async_copy(
                    inp_hbm.at[pl.ds(base, CB)], buf, psem).wait()
                for j in range(KS):
                    pltpu.async_copy(
                        atab_hbm.at[idxa.at[pl.ds(j * 128, 128)]],
                        buf.at[pl.ds(j * 128, 128)], gsem, add=True)
                for j in range(KS):
                    pltpu.async_copy(
                        ntab_hbm.at[idxr.at[pl.ds(j * 128, 128)]],
                        buf.at[pl.ds(j * 128, 128)], gsem, add=True)

        def finish(cid, bset):
            idxa, idxr, buf, psem, gsem = bset

            @pl.when((cid >= 0) & (cid < NCHUNK_B))
            def _():
                base = cid * CB
                for _ in range(2 * KS):
                    pltpu.make_async_copy(
                        inp_hbm.at[pl.ds(base, 128)],
                        buf.at[pl.ds(0, 128)], gsem).wait()

                if relu:
                    def relu_row(i, c2):
                        for j in range(8):
                            sl = pl.ds(j * LANES, LANES)
                            buf[i, sl] = jnp.maximum(buf[i, sl], 0.0)
                        return c2

                    lax.fori_loop(0, CB, relu_row, 0)

                pltpu.sync_copy(buf, out_hbm.at[pl.ds(base, CB)])

        prefetch(wid, bufs[0])

        def pair(k, carry):
            c0 = wid + (2 * k) * NW
            c1 = wid + (2 * k + 1) * NW
            fire(c0, bufs[0])
            finish(c1 - 2 * NW, bufs[1])
            prefetch(c1, bufs[1])
            fire(c1, bufs[1])
            finish(c0, bufs[0])
            prefetch(c0 + 2 * NW, bufs[0])
            return carry

        npair = (NCHUNK_B + 2 * NW - 1) // (2 * NW) + 1
        lax.fori_loop(0, npair, pair, 0)

    return pl.kernel(
        body,
        out_type=jax.ShapeDtypeStruct((N_BONDS, HIDDEN), jnp.float32),
        mesh=_sc_mesh(),
        scratch_types=[
            pltpu.VMEM((CB,), jnp.int32),
            pltpu.VMEM((CB,), jnp.int32),
            pltpu.VMEM((CB, HIDDEN), jnp.float32),
            pltpu.VMEM((CB,), jnp.int32),
            pltpu.VMEM((CB,), jnp.int32),
            pltpu.VMEM((CB, HIDDEN), jnp.float32),
            pltpu.SemaphoreType.DMA,
            pltpu.SemaphoreType.DMA,
            pltpu.SemaphoreType.DMA,
            pltpu.SemaphoreType.DMA,
        ],
    )


def _tc_init(f_bonds, W_i, W_h):
    def body(fb_ref, wi_ref, wh_ref, inp_ref, nmh_ref):
        inp = jnp.dot(fb_ref[...], wi_ref[...], preferred_element_type=jnp.float32)
        inp_ref[...] = inp
        msg = jnp.maximum(inp, 0.0)
        nmh_ref[...] = -jnp.dot(msg, wh_ref[...], preferred_element_type=jnp.float32)

    return pl.pallas_call(
        body,
        grid=(N_BONDS // BR,),
        in_specs=[
            pl.BlockSpec((BR, BOND_FDIM), lambda i: (i, 0)),
            pl.BlockSpec((BOND_FDIM, HIDDEN), lambda i: (0, 0)),
            pl.BlockSpec((HIDDEN, HIDDEN), lambda i: (0, 0)),
        ],
        out_specs=[
            pl.BlockSpec((BR, HIDDEN), lambda i: (i, 0)),
            pl.BlockSpec((BR, HIDDEN), lambda i: (i, 0)),
        ],
        out_shape=[
            jax.ShapeDtypeStruct((N_BONDS, HIDDEN), jnp.float32),
            jax.ShapeDtypeStruct((N_BONDS, HIDDEN), jnp.float32),
        ],
    )(f_bonds, W_i, W_h)


def _tc_step(premsg, W_h):
    def body(pm_ref, wh_ref, nmh_ref):
        msg = jnp.maximum(pm_ref[...], 0.0)
        nmh_ref[...] = -jnp.dot(msg, wh_ref[...], preferred_element_type=jnp.float32)

    return pl.pallas_call(
        body,
        grid=(N_BONDS // BR,),
        in_specs=[
            pl.BlockSpec((BR, HIDDEN), lambda i: (i, 0)),
            pl.BlockSpec((HIDDEN, HIDDEN), lambda i: (0, 0)),
        ],
        out_specs=pl.BlockSpec((BR, HIDDEN), lambda i: (i, 0)),
        out_shape=jax.ShapeDtypeStruct((N_BONDS, HIDDEN), jnp.float32),
    )(premsg, W_h)


def _tc_readout(f_atoms, amsg, Wo_a, Wo_m, b_o2, seg2d):
    n_blocks = N_ATOMS // DR

    def body(fa_ref, am_ref, seg_ref, woa_ref, wom_ref, bo_ref, out_ref,
             sums_v, cnts_v):
        k = pl.program_id(0)

        @pl.when(k == 0)
        def _():
            sums_v[...] = jnp.zeros_like(sums_v)
            cnts_v[...] = jnp.zeros_like(cnts_v)

        hid = jnp.dot(fa_ref[...], woa_ref[...], preferred_element_type=jnp.float32)
        hid += jnp.dot(am_ref[...], wom_ref[...], preferred_element_type=jnp.float32)
        hid = jnp.maximum(hid + bo_ref[...], 0.0)
        rows = lax.broadcasted_iota(jnp.int32, (N_MOLS, DR), 0)
        oh = (rows == seg_ref[0]).astype(jnp.float32)
        sums_v[...] += jnp.dot(oh, hid, preferred_element_type=jnp.float32)
        cnts_v[...] += jnp.sum(oh, axis=1, keepdims=True)

        @pl.when(k == n_blocks - 1)
        def _():
            out_ref[...] = sums_v[...] / jnp.maximum(cnts_v[...], 1.0)

    return pl.pallas_call(
        body,
        grid=(n_blocks,),
        in_specs=[
            pl.BlockSpec((DR, ATOM_FDIM), lambda k: (k, 0)),
            pl.BlockSpec((DR, HIDDEN), lambda k: (k, 0)),
            pl.BlockSpec((1, 1, DR), lambda k: (k, 0, 0)),
            pl.BlockSpec((ATOM_FDIM, HIDDEN), lambda k: (0, 0)),
            pl.BlockSpec((HIDDEN, HIDDEN), lambda k: (0, 0)),
            pl.BlockSpec((1, HIDDEN), lambda k: (0, 0)),
        ],
        out_specs=pl.BlockSpec((N_MOLS, HIDDEN), lambda k: (0, 0)),
        out_shape=jax.ShapeDtypeStruct((N_MOLS, HIDDEN), jnp.float32),
        scratch_shapes=[
            pltpu.VMEM((N_MOLS, HIDDEN), jnp.float32),
            pltpu.VMEM((N_MOLS, 1), jnp.float32),
        ],
    )(f_atoms, amsg, seg2d, Wo_a, Wo_m, b_o2)


def kernel(f_atoms, f_bonds, W_i, W_h, W_o, b_o, a2b, b2a, b2revb, segment_ids):
    a2b = a2b.astype(jnp.int32)
    b2a = b2a.astype(jnp.int32)
    b2revb = b2revb.astype(jnp.int32)
    segment_ids = segment_ids.astype(jnp.int32)

    n_pad = NA_PAD - N_ATOMS
    pad_block = (
        jnp.arange(n_pad * MAX_NB, dtype=jnp.int32) % N_BONDS
    ).reshape(n_pad, MAX_NB)
    a2bS = (
        jnp.concatenate([a2b, pad_block], axis=0)
        .reshape(NCHUNK_A, CA, MAX_NB)
        .transpose(0, 2, 1)
    )
    Wo_a = W_o[:ATOM_FDIM]
    Wo_m = W_o[ATOM_FDIM:]
    b_o2 = b_o.reshape(1, HIDDEN)
    seg2d = segment_ids.reshape(N_ATOMS // DR, 1, DR)

    gather_neg = _make_gather_sum(negate=True)
    gather_plain = _make_gather_sum(negate=False)
    combine = _make_combine(relu=False)
    combine_relu = _make_combine(relu=True)

    inp, nmh1 = _tc_init(f_bonds, W_i, W_h)
    a1 = gather_neg(nmh1, a2bS)
    premsg2 = combine(inp, a1, nmh1, b2a, b2revb)
    nmh2 = _tc_step(premsg2, W_h)
    a2 = gather_neg(nmh2, a2bS)
    msg3 = combine_relu(inp, a2, nmh2, b2a, b2revb)
    amsg3 = gather_plain(msg3, a2bS)
    return _tc_readout(f_atoms, amsg3[:N_ATOMS], Wo_a, Wo_m, b_o2, seg2d)


# speedup vs baseline: 2.0164x; 1.0362x over previous
import functools

import jax
import jax.numpy as jnp
from jax import lax
from jax.experimental import pallas as pl
from jax.experimental.pallas import tpu as pltpu
from jax.experimental.pallas import tpu_sc as plsc

N_ATOMS = 10000
N_BONDS = 320000
MAX_NB = 32
ATOM_FDIM = 133
BOND_FDIM = 147
HIDDEN = 128
N_MOLS = 500

NC = 2
NS = 16
NW = NC * NS
LANES = 16

NA_PAD = 10240
CA = 64
NCHUNK_A = NA_PAD // CA
NGRP = 4

CB = 256
NCHUNK_B = N_BONDS // CB
KS = CB // 128

BR = 10000
DR = 1000


def _sc_mesh():
    return plsc.VectorSubcoreMesh(
        core_axis_name="c", subcore_axis_name="s", num_cores=NC, num_subcores=NS
    )


def _make_gather_sum(negate: bool):
    def body(table_hbm, a2bS_hbm, out_hbm, idx0, acc0, idx1, acc1,
             psem0, psem1, lsem0, lsem1, gsem0, gsem1):
        wid = lax.axis_index("s") * NC + lax.axis_index("c")
        bufs = ((idx0, acc0, psem0, lsem0, gsem0),
                (idx1, acc1, psem1, lsem1, gsem1))

        def prefetch(cid, bset):
            idx, acc, psem, lsem, gsem = bset

            @pl.when((cid >= 0) & (cid < NCHUNK_A))
            def _():
                pltpu.async_copy(a2bS_hbm.at[cid], idx, psem)

        def fire(cid, bset):
            idx, acc, psem, lsem, gsem = bset

            @pl.when((cid >= 0) & (cid < NCHUNK_A))
            def _():
                pltpu.make_async_copy(a2bS_hbm.at[cid], idx, psem).wait()
                pltpu.async_copy(table_hbm.at[idx.at[0]], acc, lsem).wait()
                for j in range(MAX_NB - 1):
                    pltpu.async_copy(
                        table_hbm.at[idx.at[1 + j]], acc, gsem, add=True
                    )

        def finish(cid, bset):
            idx, acc, psem, lsem, gsem = bset

            @pl.when((cid >= 0) & (cid < NCHUNK_A))
            def _():
                abase = cid * CA
                for _ in range(MAX_NB - 1):
                    pltpu.make_async_copy(
                        out_hbm.at[pl.ds(abase, CA)], acc, gsem).wait()

                if negate:
                    def neg_row(i, c3):
                        for j in range(8):
                            sl = pl.ds(j * LANES, LANES)
                            acc[i, sl] = 0.0 - acc[i, sl]
                        return c3

                    lax.fori_loop(0, CA, neg_row, 0)

                pltpu.sync_copy(acc, out_hbm.at[pl.ds(abase, CA)])

        prefetch(wid, bufs[0])

        def pair(k, carry):
            c0 = wid + (2 * k) * NW
            c1 = wid + (2 * k + 1) * NW
            fire(c0, bufs[0])
            finish(c1 - 2 * NW, bufs[1])
            prefetch(c1, bufs[1])
            fire(c1, bufs[1])
            finish(c0, bufs[0])
            prefetch(c0 + 2 * NW, bufs[0])
            return carry

        npair = (NCHUNK_A + 2 * NW - 1) // (2 * NW) + 1
        lax.fori_loop(0, npair, pair, 0)

    return pl.kernel(
        body,
        out_type=jax.ShapeDtypeStruct((NA_PAD, HIDDEN), jnp.float32),
        mesh=_sc_mesh(),
        scratch_types=[
            pltpu.VMEM((MAX_NB, CA), jnp.int32),
            pltpu.VMEM((CA, HIDDEN), jnp.float32),
            pltpu.VMEM((MAX_NB, CA), jnp.int32),
            pltpu.VMEM((CA, HIDDEN), jnp.float32),
            pltpu.SemaphoreType.DMA,
            pltpu.SemaphoreType.DMA,
            pltpu.SemaphoreType.DMA,
            pltpu.SemaphoreType.DMA,
            pltpu.SemaphoreType.DMA,
            pltpu.SemaphoreType.DMA,
        ],
    )


def _make_combine(relu: bool):
    def body(inp_hbm, atab_hbm, ntab_hbm, b2a_hbm, b2revb_hbm, out_hbm,
             idxa0, idxr0, buf0, idxa1, idxr1, buf1, psem0, psem1,
             gsem0, gsem1):
        wid = lax.axis_index("s") * NC + lax.axis_index("c")
        bufs = ((idxa0, idxr0, buf0, psem0, gsem0),
                (idxa1, idxr1, buf1, psem1, gsem1))

        def prefetch(cid, bset):
            idxa, idxr, buf, psem, _ = bset

            @pl.when((cid >= 0) & (cid < NCHUNK_B))
            def _():
                base = cid * CB
                pltpu.async_copy(b2a_hbm.at[pl.ds(base, CB)], idxa, psem)
                pltpu.async_copy(b2revb_hbm.at[pl.ds(base, CB)], idxr, psem)
                pltpu.async_copy(inp_hbm.at[pl.ds(base, CB)], buf, psem)

        def fire(cid, bset):
            idxa, idxr, buf, psem, gsem = bset

            @pl.when((cid >= 0) & (cid < NCHUNK_B))
            def _():
                base = cid * CB
                pltpu.make_async_copy(
                    b2a_hbm.at[pl.ds(base, CB)], idxa, psem).wait()
                pltpu.make_async_copy(
                    b2revb_hbm.at[pl.ds(base, CB)], idxr, psem).wait()
                pltpu.make_async_copy(
                    inp_hbm.at[pl.ds(base, CB)], buf, psem).wait()
                for j in range(KS):
                    pltpu.async_copy(
                        atab_hbm.at[idxa.at[pl.ds(j * 128, 128)]],
                        buf.at[pl.ds(j * 128, 128)], gsem, add=True)
                for j in range(KS):
                    pltpu.async_copy(
                        ntab_hbm.at[idxr.at[pl.ds(j * 128, 128)]],
                        buf.at[pl.ds(j * 128, 128)], gsem, add=True)

        def finish(cid, bset):
            idxa, idxr, buf, psem, gsem = bset

            @pl.when((cid >= 0) & (cid < NCHUNK_B))
            def _():
                base = cid * CB
                for _ in range(2 * KS):
                    pltpu.make_async_copy(
                        inp_hbm.at[pl.ds(base, 128)],
                        buf.at[pl.ds(0, 128)], gsem).wait()

                if relu:
                    def relu_row(i, c2):
                        for j in range(8):
                            sl = pl.ds(j * LANES, LANES)
                            buf[i, sl] = jnp.maximum(buf[i, sl], 0.0)
                        return c2

                    lax.fori_loop(0, CB, relu_row, 0)

                pltpu.sync_copy(buf, out_hbm.at[pl.ds(base, CB)])

        prefetch(wid, bufs[0])

        def pair(k, carry):
            c0 = wid + (2 * k) * NW
            c1 = wid + (2 * k + 1) * NW
            fire(c0, bufs[0])
            finish(c1 - 2 * NW, bufs[1])
            prefetch(c1, bufs[1])
            fire(c1, bufs[1])
            finish(c0, bufs[0])
            prefetch(c0 + 2 * NW, bufs[0])
            return carry

        npair = (NCHUNK_B + 2 * NW - 1) // (2 * NW) + 1
        lax.fori_loop(0, npair, pair, 0)

    return pl.kernel(
        body,
        out_type=jax.ShapeDtypeStruct((N_BONDS, HIDDEN), jnp.float32),
        mesh=_sc_mesh(),
        scratch_types=[
            pltpu.VMEM((CB,), jnp.int32),
            pltpu.VMEM((CB,), jnp.int32),
            pltpu.VMEM((CB, HIDDEN), jnp.float32),
            pltpu.VMEM((CB,), jnp.int32),
            pltpu.VMEM((CB,), jnp.int32),
            pltpu.VMEM((CB, HIDDEN), jnp.float32),
            pltpu.SemaphoreType.DMA,
            pltpu.SemaphoreType.DMA,
            pltpu.SemaphoreType.DMA,
            pltpu.SemaphoreType.DMA,
        ],
    )


def _tc_init(f_bonds, W_i, W_h):
    def body(fb_ref, wi_ref, wh_ref, inp_ref, nmh_ref):
        inp = jnp.dot(fb_ref[...], wi_ref[...], preferred_element_type=jnp.float32)
        inp_ref[...] = inp
        msg = jnp.maximum(inp, 0.0)
        nmh_ref[...] = -jnp.dot(msg, wh_ref[...], preferred_element_type=jnp.float32)

    return pl.pallas_call(
        body,
        grid=(N_BONDS // BR,),
        in_specs=[
            pl.BlockSpec((BR, BOND_FDIM), lambda i: (i, 0)),
            pl.BlockSpec((BOND_FDIM, HIDDEN), lambda i: (0, 0)),
            pl.BlockSpec((HIDDEN, HIDDEN), lambda i: (0, 0)),
        ],
        out_specs=[
            pl.BlockSpec((BR, HIDDEN), lambda i: (i, 0)),
            pl.BlockSpec((BR, HIDDEN), lambda i: (i, 0)),
        ],
        out_shape=[
            jax.ShapeDtypeStruct((N_BONDS, HIDDEN), jnp.float32),
            jax.ShapeDtypeStruct((N_BONDS, HIDDEN), jnp.float32),
        ],
    )(f_bonds, W_i, W_h)


def _tc_step(premsg, W_h):
    def body(pm_ref, wh_ref, nmh_ref):
        msg = jnp.maximum(pm_ref[...], 0.0)
        nmh_ref[...] = -jnp.dot(msg, wh_ref[...], preferred_element_type=jnp.float32)

    return pl.pallas_call(
        body,
        grid=(N_BONDS // BR,),
        in_specs=[
            pl.BlockSpec((BR, HIDDEN), lambda i: (i, 0)),
            pl.BlockSpec((HIDDEN, HIDDEN), lambda i: (0, 0)),
        ],
        out_specs=pl.BlockSpec((BR, HIDDEN), lambda i: (i, 0)),
        out_shape=jax.ShapeDtypeStruct((N_BONDS, HIDDEN), jnp.float32),
    )(premsg, W_h)


def _tc_readout(f_atoms, amsg, Wo_a, Wo_m, b_o2, seg2d):
    n_blocks = N_ATOMS // DR

    def body(fa_ref, am_ref, seg_ref, woa_ref, wom_ref, bo_ref, out_ref,
             sums_v, cnts_v):
        k = pl.program_id(0)

        @pl.when(k == 0)
        def _():
            sums_v[...] = jnp.zeros_like(sums_v)
            cnts_v[...] = jnp.zeros_like(cnts_v)

        hid = jnp.dot(fa_ref[...], woa_ref[...], preferred_element_type=jnp.float32)
        hid += jnp.dot(am_ref[...], wom_ref[...], preferred_element_type=jnp.float32)
        hid = jnp.maximum(hid + bo_ref[...], 0.0)
        rows = lax.broadcasted_iota(jnp.int32, (N_MOLS, DR), 0)
        oh = (rows == seg_ref[0]).astype(jnp.float32)
        sums_v[...] += jnp.dot(oh, hid, preferred_element_type=jnp.float32)
        cnts_v[...] += jnp.sum(oh, axis=1, keepdims=True)

        @pl.when(k == n_blocks - 1)
        def _():
            out_ref[...] = sums_v[...] / jnp.maximum(cnts_v[...], 1.0)

    return pl.pallas_call(
        body,
        grid=(n_blocks,),
        in_specs=[
            pl.BlockSpec((DR, ATOM_FDIM), lambda k: (k, 0)),
            pl.BlockSpec((DR, HIDDEN), lambda k: (k, 0)),
            pl.BlockSpec((1, 1, DR), lambda k: (k, 0, 0)),
            pl.BlockSpec((ATOM_FDIM, HIDDEN), lambda k: (0, 0)),
            pl.BlockSpec((HIDDEN, HIDDEN), lambda k: (0, 0)),
            pl.BlockSpec((1, HIDDEN), lambda k: (0, 0)),
        ],
        out_specs=pl.BlockSpec((N_MOLS, HIDDEN), lambda k: (0, 0)),
        out_shape=jax.ShapeDtypeStruct((N_MOLS, HIDDEN), jnp.float32),
        scratch_shapes=[
            pltpu.VMEM((N_MOLS, HIDDEN), jnp.float32),
            pltpu.VMEM((N_MOLS, 1), jnp.float32),
        ],
    )(f_atoms, amsg, seg2d, Wo_a, Wo_m, b_o2)


def kernel(f_atoms, f_bonds, W_i, W_h, W_o, b_o, a2b, b2a, b2revb, segment_ids):
    a2b = a2b.astype(jnp.int32)
    b2a = b2a.astype(jnp.int32)
    b2revb = b2revb.astype(jnp.int32)
    segment_ids = segment_ids.astype(jnp.int32)

    n_pad = NA_PAD - N_ATOMS
    pad_block = (
        jnp.arange(n_pad * MAX_NB, dtype=jnp.int32) % N_BONDS
    ).reshape(n_pad, MAX_NB)
    a2bS = (
        jnp.concatenate([a2b, pad_block], axis=0)
        .reshape(NCHUNK_A, CA, MAX_NB)
        .transpose(0, 2, 1)
    )
    Wo_a = W_o[:ATOM_FDIM]
    Wo_m = W_o[ATOM_FDIM:]
    b_o2 = b_o.reshape(1, HIDDEN)
    seg2d = segment_ids.reshape(N_ATOMS // DR, 1, DR)

    gather_neg = _make_gather_sum(negate=True)
    gather_plain = _make_gather_sum(negate=False)
    combine = _make_combine(relu=False)
    combine_relu = _make_combine(relu=True)

    inp, nmh1 = _tc_init(f_bonds, W_i, W_h)
    a1 = gather_neg(nmh1, a2bS)
    premsg2 = combine(inp, a1, nmh1, b2a, b2revb)
    nmh2 = _tc_step(premsg2, W_h)
    a2 = gather_neg(nmh2, a2bS)
    msg3 = combine_relu(inp, a2, nmh2, b2a, b2revb)
    amsg3 = gather_plain(msg3, a2bS)
    return _tc_readout(f_atoms, amsg3[:N_ATOMS], Wo_a, Wo_m, b_o2, seg2d)
